# baseline (speedup 1.0000x reference)
import functools

import jax
import jax.numpy as jnp
from jax import lax
from jax.experimental import pallas as pl
from jax.experimental.pallas import tpu as pltpu
from jax.experimental.pallas import tpu_sc as plsc

N = 4194304
M = 2097152
_CHUNK = 4096
_LANES = 16


@functools.cache
def _build():
    info = plsc.get_sparse_core_info()
    nc, ns = info.num_cores, info.num_subcores
    nw = nc * ns
    per_w = M // nw
    n_chunks = per_w // _CHUNK

    mesh = plsc.VectorSubcoreMesh(core_axis_name="c", subcore_axis_name="s")

    @functools.partial(
        pl.kernel,
        mesh=mesh,
        out_type=(
            jax.ShapeDtypeStruct((M,), jnp.int32),
            jax.ShapeDtypeStruct((2 * M,), jnp.float32),
            jax.ShapeDtypeStruct((M,), jnp.float32),
        ),
        scratch_types=[
            pltpu.VMEM((_CHUNK,), jnp.int32),
            pltpu.VMEM((2 * _CHUNK,), jnp.int32),
            pltpu.VMEM((_CHUNK,), jnp.int32),
            pltpu.VMEM((2 * _CHUNK,), jnp.float32),
            pltpu.VMEM((_CHUNK,), jnp.float32),
            pltpu.SemaphoreType.DMA,
            pltpu.SemaphoreType.DMA,
            pltpu.SemaphoreType.DMA,
        ],
        compiler_params=pltpu.CompilerParams(use_tc_tiling_on_sc=False,
                                              needs_layout_passes=False),
    )
    def gather_kernel(index_hbm, points_hbm, depth_hbm, fidx_hbm,
                      out_idx_hbm, out_pts_hbm, out_dep_hbm,
                      idx_v, pidx_v, oi_v, op_v, od_v, sem_i, sem_p, sem_d):
        wid = lax.axis_index("s") * nc + lax.axis_index("c")
        base = wid * per_w
        lane = lax.iota(jnp.int32, _LANES)

        def body(ci, carry):
            off = base + ci * _CHUNK
            pltpu.sync_copy(fidx_hbm.at[pl.ds(off, _CHUNK)], idx_v)
            cp_i = pltpu.async_copy(index_hbm.at[idx_v], oi_v, sem_i)
            cp_d = pltpu.async_copy(depth_hbm.at[idx_v], od_v, sem_d)

            def ibody(k, carry2):
                f2 = idx_v[pl.ds(k * _LANES, _LANES)] * 2
                pos = k * (2 * _LANES) + lane * 2
                plsc.store_scatter(pidx_v, [pos], f2)
                plsc.store_scatter(pidx_v, [pos + 1], f2 + 1)
                return carry2

            lax.fori_loop(0, _CHUNK // _LANES, ibody, 0)
            cp_p = pltpu.async_copy(points_hbm.at[pidx_v], op_v, sem_p)
            cp_i.wait()
            cp_d.wait()
            pltpu.sync_copy(oi_v, out_idx_hbm.at[pl.ds(off, _CHUNK)])
            pltpu.sync_copy(od_v, out_dep_hbm.at[pl.ds(off, _CHUNK)])
            cp_p.wait()
            pltpu.sync_copy(op_v, out_pts_hbm.at[pl.ds(2 * off, 2 * _CHUNK)])
            return carry

        lax.fori_loop(0, n_chunks, body, 0)

    return gather_kernel


def kernel(index, points, depth, frame_indices):
    out_idx, out_pts_flat, out_dep = _build()(
        index, points.reshape(2 * N), depth, frame_indices)
    return (out_idx, out_pts_flat.reshape(M, 2), out_dep)


# speedup vs baseline: 21.3085x; 21.3085x over previous
import functools

import jax
import jax.numpy as jnp
from jax import lax
from jax.experimental import pallas as pl
from jax.experimental.pallas import tpu as pltpu
from jax.experimental.pallas import tpu_sc as plsc

N = 4194304
M = 2097152
_CHUNK = 4096
_LANES = 16


@functools.cache
def _build():
    info = plsc.get_sparse_core_info()
    nc, ns = info.num_cores, info.num_subcores
    nw = nc * ns
    per_w = M // nw
    n_chunks = per_w // _CHUNK

    mesh = plsc.VectorSubcoreMesh(core_axis_name="c", subcore_axis_name="s")

    @functools.partial(
        pl.kernel,
        mesh=mesh,
        out_type=(
            jax.ShapeDtypeStruct((M,), jnp.int32),
            jax.ShapeDtypeStruct((2 * M,), jnp.float32),
            jax.ShapeDtypeStruct((M,), jnp.float32),
        ),
        scratch_types=[
            pltpu.VMEM((_CHUNK,), jnp.int32),
            pltpu.VMEM((2 * _CHUNK,), jnp.int32),
            pltpu.VMEM((_CHUNK,), jnp.int32),
            pltpu.VMEM((2 * _CHUNK,), jnp.float32),
            pltpu.VMEM((_CHUNK,), jnp.float32),
            pltpu.SemaphoreType.DMA,
            pltpu.SemaphoreType.DMA,
            pltpu.SemaphoreType.DMA,
        ],
        compiler_params=pltpu.CompilerParams(use_tc_tiling_on_sc=False,
                                             needs_layout_passes=False),
    )
    def gather_kernel(index_hbm, points_hbm, depth_hbm, fidx_hbm,
                      out_idx_hbm, out_pts_hbm, out_dep_hbm,
                      idx_v, pidx_v, oi_v, op_v, od_v, sem_i, sem_p, sem_d):
        wid = lax.axis_index("s") * nc + lax.axis_index("c")
        base = wid * per_w

        def body(ci, carry):
            off = base + ci * _CHUNK
            pltpu.sync_copy(fidx_hbm.at[pl.ds(off, _CHUNK)], idx_v)
            cp_i = pltpu.async_copy(index_hbm.at[idx_v], oi_v, sem_i)
            cp_d = pltpu.async_copy(depth_hbm.at[idx_v], od_v, sem_d)

            def ibody(k, carry2):
                f = idx_v[pl.ds(k * _LANES, _LANES)]
                px = f * 2 - lax.rem(f, 128)
                dst = (k // 8) * 256 + (k % 8) * _LANES
                pidx_v[pl.ds(dst, _LANES)] = px
                pidx_v[pl.ds(dst + 128, _LANES)] = px + 128
                return carry2

            lax.fori_loop(0, _CHUNK // _LANES, ibody, 0)
            cp_p = pltpu.async_copy(points_hbm.at[pidx_v], op_v, sem_p)
            cp_i.wait()
            cp_d.wait()
            pltpu.sync_copy(oi_v, out_idx_hbm.at[pl.ds(off, _CHUNK)])
            pltpu.sync_copy(od_v, out_dep_hbm.at[pl.ds(off, _CHUNK)])
            cp_p.wait()
            pltpu.sync_copy(op_v, out_pts_hbm.at[pl.ds(2 * off, 2 * _CHUNK)])
            return carry

        lax.fori_loop(0, n_chunks, body, 0)

    return gather_kernel


def kernel(index, points, depth, frame_indices):
    pts_blocked = points.reshape(N // 128, 128, 2).transpose(0, 2, 1)
    out_idx, out_pts_flat, out_dep = _build()(
        index, pts_blocked.reshape(2 * N), depth, frame_indices)
    out_pts = (out_pts_flat.reshape(M // 128, 2, 128)
               .transpose(0, 2, 1).reshape(M, 2))
    return (out_idx, out_pts, out_dep)


# speedup vs baseline: 21.3167x; 1.0004x over previous
import functools

import jax
import jax.numpy as jnp
from jax import lax
from jax.experimental import pallas as pl
from jax.experimental.pallas import tpu as pltpu
from jax.experimental.pallas import tpu_sc as plsc

N = 4194304
M = 2097152
_CHUNK = 4096
_LANES = 16


@functools.cache
def _build():
    info = plsc.get_sparse_core_info()
    nc, ns = info.num_cores, info.num_subcores
    nw = nc * ns
    per_w = M // nw
    n_chunks = per_w // _CHUNK

    mesh = plsc.VectorSubcoreMesh(core_axis_name="c", subcore_axis_name="s")

    @functools.partial(
        pl.kernel,
        mesh=mesh,
        out_type=(
            jax.ShapeDtypeStruct((M,), jnp.int32),
            jax.ShapeDtypeStruct((2 * M,), jnp.float32),
            jax.ShapeDtypeStruct((M,), jnp.float32),
        ),
        scratch_types=[
            pltpu.VMEM((2, _CHUNK), jnp.int32),
            pltpu.VMEM((2, 2 * _CHUNK), jnp.int32),
            pltpu.VMEM((2, _CHUNK), jnp.int32),
            pltpu.VMEM((2, 2 * _CHUNK), jnp.float32),
            pltpu.VMEM((2, _CHUNK), jnp.float32),
            pltpu.SemaphoreType.DMA,
            pltpu.SemaphoreType.DMA,
            pltpu.SemaphoreType.DMA,
        ],
        compiler_params=pltpu.CompilerParams(use_tc_tiling_on_sc=False,
                                             needs_layout_passes=False),
    )
    def gather_kernel(index_hbm, points_hbm, depth_hbm, fidx_hbm,
                      out_idx_hbm, out_pts_hbm, out_dep_hbm,
                      idx_v, pidx_v, oi_v, op_v, od_v, sem_i, sem_p, sem_d):
        wid = lax.axis_index("s") * nc + lax.axis_index("c")
        base = wid * per_w

        def stage_and_fire(ci, b):
            off = base + ci * _CHUNK
            pltpu.sync_copy(fidx_hbm.at[pl.ds(off, _CHUNK)], idx_v.at[b])
            pltpu.async_copy(index_hbm.at[idx_v.at[b]], oi_v.at[b], sem_i)
            pltpu.async_copy(depth_hbm.at[idx_v.at[b]], od_v.at[b], sem_d)

            def ibody(k, carry2):
                f = idx_v[b, pl.ds(k * _LANES, _LANES)]
                px = f * 2 - lax.rem(f, 128)
                dst = (k // 8) * 256 + (k % 8) * _LANES
                pidx_v[b, pl.ds(dst, _LANES)] = px
                pidx_v[b, pl.ds(dst + 128, _LANES)] = px + 128
                return carry2

            lax.fori_loop(0, _CHUNK // _LANES, ibody, 0)
            pltpu.async_copy(points_hbm.at[pidx_v.at[b]], op_v.at[b], sem_p)

        def drain_and_write(ci, b):
            off = base + ci * _CHUNK
            pltpu.make_async_copy(index_hbm.at[pl.ds(0, _CHUNK)],
                                  oi_v.at[b], sem_i).wait()
            pltpu.make_async_copy(depth_hbm.at[pl.ds(0, _CHUNK)],
                                  od_v.at[b], sem_d).wait()
            pltpu.sync_copy(oi_v.at[b], out_idx_hbm.at[pl.ds(off, _CHUNK)])
            pltpu.sync_copy(od_v.at[b], out_dep_hbm.at[pl.ds(off, _CHUNK)])
            pltpu.make_async_copy(points_hbm.at[pl.ds(0, 2 * _CHUNK)],
                                  op_v.at[b], sem_p).wait()
            pltpu.sync_copy(op_v.at[b],
                            out_pts_hbm.at[pl.ds(2 * off, 2 * _CHUNK)])

        stage_and_fire(0, 0)

        def body(ci, carry):
            b = lax.rem(ci, 2)
            stage_and_fire(ci, b)
            drain_and_write(ci - 1, 1 - b)
            return carry

        lax.fori_loop(1, n_chunks, body, 0)
        drain_and_write(n_chunks - 1, lax.rem(n_chunks - 1, 2))

    return gather_kernel


def kernel(index, points, depth, frame_indices):
    pts_blocked = points.reshape(N // 128, 128, 2).transpose(0, 2, 1)
    out_idx, out_pts_flat, out_dep = _build()(
        index, pts_blocked.reshape(2 * N), depth, frame_indices)
    out_pts = (out_pts_flat.reshape(M // 128, 2, 128)
               .transpose(0, 2, 1).reshape(M, 2))
    return (out_idx, out_pts, out_dep)
